# baseline (speedup 1.0000x reference)
import numpy as np
import jax
import jax.numpy as jnp
from jax.experimental import pallas as pl
from jax.experimental.pallas import tpu as pltpu

N_SPECIES = 20
EMB_DIM = 64
N_RBF = 16
N_CONV = 3
A = 24
PAIRS = A * A
BLK = 8

_CENTERS = np.linspace(0.0, 5.0, N_RBF).astype(np.float32)


def _softplus(v):
    return jnp.maximum(v, 0.0) + jnp.log1p(jnp.exp(-jnp.abs(v)))


def _mpinn_kernel(ppos_ref, spec_ref, emb_ref, Wc1_ref, bc1_ref, Wc2_ref,
                  bc2_ref, Wu1_ref, bu1_ref, Wu2_ref, bu2_ref, Wp1_ref,
                  bp1_ref, wp2_ref, bp2_ref, out_ref):
    ppos = ppos_ref[...]
    diff = ppos[:, 0:3] - ppos[:, 3:6]
    d2 = jnp.sum(diff * diff, axis=1, keepdims=True)
    dist = jnp.sqrt(d2 + 1e-12)
    centers = jax.lax.broadcasted_iota(
        jnp.int32, (1, N_RBF), 1).astype(jnp.float32) * (5.0 / (N_RBF - 1))
    rbf = jnp.exp(-2.0 * (dist - centers) ** 2)
    g = _softplus(jnp.dot(rbf, Wc1_ref[...],
                          preferred_element_type=jnp.float32) + bc1_ref[...])
    f = jnp.dot(g, Wc2_ref[...],
                preferred_element_type=jnp.float32) + bc2_ref[...]
    rows = f.shape[0]
    p = jax.lax.broadcasted_iota(jnp.int32, (rows, 1), 0) % PAIRS
    f = jnp.where((p % (A + 1)) == 0, 0.0, f)

    spec = spec_ref[...]
    oh = (spec == jax.lax.broadcasted_iota(
        jnp.int32, (spec.shape[0], N_SPECIES), 1)).astype(jnp.float32)
    h = jnp.dot(oh, emb_ref[...], preferred_element_type=jnp.float32)

    f4 = f.reshape(BLK, A, A, EMB_DIM)
    for _ in range(N_CONV):
        hj = h.reshape(BLK, 1, A, EMB_DIM)
        msg = jnp.sum(f4 * hj, axis=2).reshape(BLK * A, EMB_DIM)
        u = _softplus(jnp.dot(msg, Wu1_ref[...],
                              preferred_element_type=jnp.float32) + bu1_ref[...])
        u = jnp.dot(u, Wu2_ref[...],
                    preferred_element_type=jnp.float32) + bu2_ref[...]
        h = h + u

    node = _softplus(jnp.dot(h, Wp1_ref[...],
                             preferred_element_type=jnp.float32) + bp1_ref[...])
    val = jnp.sum(node * wp2_ref[...], axis=1, keepdims=True) + bp2_ref[...]
    out_ref[...] = jnp.sum(val.reshape(BLK, A, 1), axis=1)


def kernel(x, emb, Wc1, bc1, Wc2, bc2, Wu1, bu1, Wu2, bu2, Wp1, bp1, Wp2, bp2):
    B = x.shape[0]
    species = x[..., 0].astype(jnp.int32).reshape(B * A, 1)
    pos = x[..., 1:4]
    pi = jnp.broadcast_to(pos[:, :, None, :], (B, A, A, 3)).reshape(B * PAIRS, 3)
    pj = jnp.broadcast_to(pos[:, None, :, :], (B, A, A, 3)).reshape(B * PAIRS, 3)
    ppos = jnp.concatenate([pi, pj], axis=1)

    nb = B // BLK
    full = lambda a: pl.BlockSpec(a.shape, lambda i: (0,) * a.ndim)
    out = pl.pallas_call(
        _mpinn_kernel,
        grid=(nb,),
        in_specs=[
            pl.BlockSpec((BLK * PAIRS, 6), lambda i: (i, 0)),
            pl.BlockSpec((BLK * A, 1), lambda i: (i, 0)),
            full(emb), full(Wc1), full(bc1.reshape(1, -1)), full(Wc2),
            full(bc2.reshape(1, -1)), full(Wu1), full(bu1.reshape(1, -1)),
            full(Wu2), full(bu2.reshape(1, -1)), full(Wp1),
            full(bp1.reshape(1, -1)), full(Wp2.reshape(1, EMB_DIM)),
            full(bp2.reshape(1, 1)),
        ],
        out_specs=pl.BlockSpec((BLK, 1), lambda i: (i, 0)),
        out_shape=jax.ShapeDtypeStruct((B, 1), jnp.float32),
        compiler_params=pltpu.CompilerParams(
            dimension_semantics=("arbitrary",)),
    )(ppos, species, emb, Wc1, bc1.reshape(1, -1), Wc2, bc2.reshape(1, -1),
      Wu1, bu1.reshape(1, -1), Wu2, bu2.reshape(1, -1), Wp1,
      bp1.reshape(1, -1), Wp2.reshape(1, EMB_DIM), bp2.reshape(1, 1))
    return out.reshape(B)


# speedup vs baseline: 2.2062x; 2.2062x over previous
import numpy as np
import jax
import jax.numpy as jnp
from jax.experimental import pallas as pl
from jax.experimental.pallas import tpu as pltpu

N_SPECIES = 20
EMB_DIM = 64
N_RBF = 16
N_CONV = 3
A = 24
PAIRS = A * A
BLK = 8
PACK = BLK * PAIRS

_DN_KM = (((0,), (0,)), ((), ()))


def _softplus(v):
    return jnp.maximum(v, 0.0) + jnp.log1p(jnp.exp(-jnp.abs(v)))


def _mpinn_kernel(pi_ref, pj_ref, spec_ref, emb_ref, Wc1_ref, bc1_ref,
                  Wc2_ref, bc2_ref, Wu1_ref, bu1_ref, Wu2_ref, bu2_ref,
                  Wp1_ref, bp1_ref, wp2_ref, bp2_ref, out_ref):
    d = pi_ref[0] - pj_ref[0]
    d2 = jnp.sum(d * d, axis=0, keepdims=True)
    dist = jnp.sqrt(d2 + 1e-12)
    centers = jax.lax.broadcasted_iota(
        jnp.int32, (N_RBF, 1), 0).astype(jnp.float32) * (5.0 / (N_RBF - 1))
    rbfT = jnp.exp(-2.0 * (dist - centers) ** 2)
    gT = _softplus(jax.lax.dot_general(
        Wc1_ref[...], rbfT, _DN_KM,
        preferred_element_type=jnp.float32) + bc1_ref[...])
    f = jax.lax.dot_general(gT, Wc2_ref[...], _DN_KM,
                            preferred_element_type=jnp.float32)
    f = f + bc2_ref[...]
    p = jax.lax.broadcasted_iota(jnp.int32, (PACK, 1), 0) % PAIRS
    f = jnp.where((p % (A + 1)) == 0, 0.0, f)

    spec = spec_ref[...]
    oh = (spec == jax.lax.broadcasted_iota(
        jnp.int32, (spec.shape[0], N_SPECIES), 1)).astype(jnp.float32)
    h = jnp.dot(oh, emb_ref[...], preferred_element_type=jnp.float32)

    f4 = f.reshape(BLK, A, A, EMB_DIM)
    for _ in range(N_CONV):
        hj = h.reshape(BLK, 1, A, EMB_DIM)
        msg = jnp.sum(f4 * hj, axis=2).reshape(BLK * A, EMB_DIM)
        u = _softplus(jnp.dot(msg, Wu1_ref[...],
                              preferred_element_type=jnp.float32) + bu1_ref[...])
        u = jnp.dot(u, Wu2_ref[...],
                    preferred_element_type=jnp.float32) + bu2_ref[...]
        h = h + u

    node = _softplus(jnp.dot(h, Wp1_ref[...],
                             preferred_element_type=jnp.float32) + bp1_ref[...])
    val = jnp.sum(node * wp2_ref[...], axis=1, keepdims=True) + bp2_ref[...]
    out_ref[...] = jnp.sum(val.reshape(BLK, A, 1), axis=1)


def kernel(x, emb, Wc1, bc1, Wc2, bc2, Wu1, bu1, Wu2, bu2, Wp1, bp1, Wp2, bp2):
    B = x.shape[0]
    nb = B // BLK
    species = x[..., 0].astype(jnp.int32).reshape(B * A, 1)
    pos = x[..., 1:4]

    def planes(p4):
        return (p4.transpose(0, 3, 1, 2).reshape(nb, BLK, 3, PAIRS)
                .transpose(0, 2, 1, 3).reshape(nb, 3, PACK))

    Pi = planes(jnp.broadcast_to(pos[:, :, None, :], (B, A, A, 3)))
    Pj = planes(jnp.broadcast_to(pos[:, None, :, :], (B, A, A, 3)))

    full = lambda a: pl.BlockSpec(a.shape, lambda i: (0,) * a.ndim)
    out = pl.pallas_call(
        _mpinn_kernel,
        grid=(nb,),
        in_specs=[
            pl.BlockSpec((1, 3, PACK), lambda i: (i, 0, 0)),
            pl.BlockSpec((1, 3, PACK), lambda i: (i, 0, 0)),
            pl.BlockSpec((BLK * A, 1), lambda i: (i, 0)),
            full(emb), full(Wc1), full(bc1.reshape(-1, 1)), full(Wc2),
            full(bc2.reshape(1, -1)), full(Wu1), full(bu1.reshape(1, -1)),
            full(Wu2), full(bu2.reshape(1, -1)), full(Wp1),
            full(bp1.reshape(1, -1)), full(Wp2.reshape(1, EMB_DIM)),
            full(bp2.reshape(1, 1)),
        ],
        out_specs=pl.BlockSpec((BLK, 1), lambda i: (i, 0)),
        out_shape=jax.ShapeDtypeStruct((B, 1), jnp.float32),
        compiler_params=pltpu.CompilerParams(
            dimension_semantics=("parallel",)),
    )(Pi, Pj, species, emb, Wc1, bc1.reshape(-1, 1), Wc2, bc2.reshape(1, -1),
      Wu1, bu1.reshape(1, -1), Wu2, bu2.reshape(1, -1), Wp1,
      bp1.reshape(1, -1), Wp2.reshape(1, EMB_DIM), bp2.reshape(1, 1))
    return out.reshape(B)


# speedup vs baseline: 2.4128x; 1.0936x over previous
import numpy as np
import jax
import jax.numpy as jnp
from jax.experimental import pallas as pl
from jax.experimental.pallas import tpu as pltpu

N_SPECIES = 20
EMB_DIM = 64
N_RBF = 16
N_CONV = 3
A = 24
PAIRS = A * A
BLK = 16
PACK = BLK * PAIRS

_DN_KM = (((0,), (0,)), ((), ()))


def _softplus(v):
    return jnp.maximum(v, 0.0) + jnp.log1p(jnp.exp(-jnp.abs(v)))


def _mpinn_kernel(pi_ref, pj_ref, spec_ref, emb_ref, Wc1_ref, bc1_ref,
                  Wc2_ref, bc2_ref, Wu1_ref, bu1_ref, Wu2_ref, bu2_ref,
                  Wp1_ref, bp1_ref, wp2_ref, bp2_ref, out_ref):
    d = pi_ref[0] - pj_ref[0]
    d2 = jnp.sum(d * d, axis=0, keepdims=True)
    dist = jnp.sqrt(d2 + 1e-12)
    centers = jax.lax.broadcasted_iota(
        jnp.int32, (N_RBF, 1), 0).astype(jnp.float32) * (5.0 / (N_RBF - 1))
    rbfT = jnp.exp(-2.0 * (dist - centers) ** 2)
    gT = _softplus(jax.lax.dot_general(
        Wc1_ref[...], rbfT, _DN_KM,
        preferred_element_type=jnp.float32) + bc1_ref[...])
    f = jax.lax.dot_general(gT, Wc2_ref[...], _DN_KM,
                            preferred_element_type=jnp.float32)
    f = f + bc2_ref[...]
    p = jax.lax.broadcasted_iota(jnp.int32, (PACK, 1), 0) % PAIRS
    f = jnp.where((p % (A + 1)) == 0, 0.0, f)

    spec = spec_ref[...]
    oh = (spec == jax.lax.broadcasted_iota(
        jnp.int32, (spec.shape[0], N_SPECIES), 1)).astype(jnp.float32)
    h = jnp.dot(oh, emb_ref[...], preferred_element_type=jnp.float32)

    f4 = f.reshape(BLK, A, A, EMB_DIM)
    for _ in range(N_CONV):
        hj = h.reshape(BLK, 1, A, EMB_DIM)
        msg = jnp.sum(f4 * hj, axis=2).reshape(BLK * A, EMB_DIM)
        u = _softplus(jnp.dot(msg, Wu1_ref[...],
                              preferred_element_type=jnp.float32) + bu1_ref[...])
        u = jnp.dot(u, Wu2_ref[...],
                    preferred_element_type=jnp.float32) + bu2_ref[...]
        h = h + u

    node = _softplus(jnp.dot(h, Wp1_ref[...],
                             preferred_element_type=jnp.float32) + bp1_ref[...])
    val = jnp.sum(node * wp2_ref[...], axis=1, keepdims=True) + bp2_ref[...]
    out_ref[...] = jnp.sum(val.reshape(BLK, A, 1), axis=1)


def kernel(x, emb, Wc1, bc1, Wc2, bc2, Wu1, bu1, Wu2, bu2, Wp1, bp1, Wp2, bp2):
    B = x.shape[0]
    nb = B // BLK
    species = x[..., 0].astype(jnp.int32).reshape(B * A, 1)
    pos = x[..., 1:4]

    def planes(p4):
        return (p4.transpose(0, 3, 1, 2).reshape(nb, BLK, 3, PAIRS)
                .transpose(0, 2, 1, 3).reshape(nb, 3, PACK))

    Pi = planes(jnp.broadcast_to(pos[:, :, None, :], (B, A, A, 3)))
    Pj = planes(jnp.broadcast_to(pos[:, None, :, :], (B, A, A, 3)))

    full = lambda a: pl.BlockSpec(a.shape, lambda i: (0,) * a.ndim)
    out = pl.pallas_call(
        _mpinn_kernel,
        grid=(nb,),
        in_specs=[
            pl.BlockSpec((1, 3, PACK), lambda i: (i, 0, 0)),
            pl.BlockSpec((1, 3, PACK), lambda i: (i, 0, 0)),
            pl.BlockSpec((BLK * A, 1), lambda i: (i, 0)),
            full(emb), full(Wc1), full(bc1.reshape(-1, 1)), full(Wc2),
            full(bc2.reshape(1, -1)), full(Wu1), full(bu1.reshape(1, -1)),
            full(Wu2), full(bu2.reshape(1, -1)), full(Wp1),
            full(bp1.reshape(1, -1)), full(Wp2.reshape(1, EMB_DIM)),
            full(bp2.reshape(1, 1)),
        ],
        out_specs=pl.BlockSpec((BLK, 1), lambda i: (i, 0)),
        out_shape=jax.ShapeDtypeStruct((B, 1), jnp.float32),
        compiler_params=pltpu.CompilerParams(
            dimension_semantics=("parallel",)),
    )(Pi, Pj, species, emb, Wc1, bc1.reshape(-1, 1), Wc2, bc2.reshape(1, -1),
      Wu1, bu1.reshape(1, -1), Wu2, bu2.reshape(1, -1), Wp1,
      bp1.reshape(1, -1), Wp2.reshape(1, EMB_DIM), bp2.reshape(1, 1))
    return out.reshape(B)


# speedup vs baseline: 4.1755x; 1.7306x over previous
import numpy as np
import jax
import jax.numpy as jnp
from jax.experimental import pallas as pl
from jax.experimental.pallas import tpu as pltpu

N_SPECIES = 20
EMB_DIM = 64
N_RBF = 16
N_CONV = 3
A = 24
PAIRS = A * A
APAD = 128
PPAD = 640
BLK = 32
PACKP = BLK * PPAD

_DN_KM = (((0,), (0,)), ((), ()))


def _softplus(v):
    return jnp.maximum(v, 0.0) + jnp.log1p(jnp.exp(-jnp.abs(v)))


def _mpinn_kernel(post_ref, sd_ref, mask_ref, spec_ref, emb_ref, Wc1_ref,
                  bc1_ref, Wc2_ref, bc2_ref, Wu1_ref, bu1_ref, Wu2_ref,
                  bu2_ref, Wp1_ref, bp1_ref, wp2_ref, bp2_ref, out_ref):
    post = post_ref[:, 0, 0, :]
    sd = sd_ref[...]
    d = jnp.concatenate(
        [jnp.dot(post[:, m * APAD:(m + 1) * APAD], sd,
                 preferred_element_type=jnp.float32) for m in range(BLK)],
        axis=1)
    d2 = jnp.sum(d * d, axis=0, keepdims=True)
    dist = jnp.sqrt(d2 + 1e-12)
    centers = jax.lax.broadcasted_iota(
        jnp.int32, (N_RBF, 1), 0).astype(jnp.float32) * (5.0 / (N_RBF - 1))
    rbfT = jnp.exp(-2.0 * (dist - centers) ** 2)
    gT = _softplus(jax.lax.dot_general(
        Wc1_ref[...], rbfT, _DN_KM,
        preferred_element_type=jnp.float32) + bc1_ref[...])
    gT = gT * mask_ref[...]
    f = jax.lax.dot_general(gT, Wc2_ref[...], _DN_KM,
                            preferred_element_type=jnp.float32)

    spec = spec_ref[...]
    oh = (spec == jax.lax.broadcasted_iota(
        jnp.int32, (spec.shape[0], N_SPECIES), 1)).astype(jnp.float32)
    h = jnp.dot(oh, emb_ref[...], preferred_element_type=jnp.float32)

    f4 = (f.reshape(BLK, PPAD, EMB_DIM)[:, :PAIRS, :]
          .reshape(BLK, A, A, EMB_DIM))
    for _ in range(N_CONV):
        h4 = h.reshape(BLK, A, EMB_DIM)
        hj = h4.reshape(BLK, 1, A, EMB_DIM)
        msg4 = jnp.sum(f4 * hj, axis=2)
        hs = jnp.sum(h4, axis=1, keepdims=True)
        msg = (msg4 + bc2_ref[...] * (hs - h4)).reshape(BLK * A, EMB_DIM)
        u = _softplus(jnp.dot(msg, Wu1_ref[...],
                              preferred_element_type=jnp.float32) + bu1_ref[...])
        u = jnp.dot(u, Wu2_ref[...],
                    preferred_element_type=jnp.float32) + bu2_ref[...]
        h = h + u

    node = _softplus(jnp.dot(h, Wp1_ref[...],
                             preferred_element_type=jnp.float32) + bp1_ref[...])
    val = jnp.sum(node * wp2_ref[...], axis=1, keepdims=True) + bp2_ref[...]
    out_ref[...] = jnp.sum(val.reshape(BLK, A, 1), axis=1)


def kernel(x, emb, Wc1, bc1, Wc2, bc2, Wu1, bu1, Wu2, bu2, Wp1, bp1, Wp2, bp2):
    B = x.shape[0]
    nb = B // BLK
    species = x[..., 0].astype(jnp.int32).reshape(B * A, 1)
    pos = x[..., 1:4]

    post = jnp.pad(pos.transpose(2, 0, 1),
                   ((0, 0), (0, 0), (0, APAD - A)))
    post = post.reshape(3, nb, 1, BLK * APAD)

    pidx = jnp.arange(PPAD, dtype=jnp.int32)
    ii = jnp.where(pidx < PAIRS, pidx // A, A)
    jj = jnp.where(pidx < PAIRS, pidx % A, A)
    aa = jnp.arange(APAD, dtype=jnp.int32)[:, None]
    sd = ((aa == ii[None, :]).astype(jnp.float32)
          - (aa == jj[None, :]).astype(jnp.float32))

    mvec = jnp.where((pidx < PAIRS) & (pidx % (A + 1) != 0), 1.0, 0.0)
    mask = jnp.tile(mvec, BLK).reshape(1, PACKP).astype(jnp.float32)

    full = lambda a: pl.BlockSpec(a.shape, lambda i: (0,) * a.ndim)
    out = pl.pallas_call(
        _mpinn_kernel,
        grid=(nb,),
        in_specs=[
            pl.BlockSpec((3, 1, 1, BLK * APAD), lambda i: (0, i, 0, 0)),
            full(sd),
            full(mask),
            pl.BlockSpec((BLK * A, 1), lambda i: (i, 0)),
            full(emb), full(Wc1), full(bc1.reshape(-1, 1)), full(Wc2),
            full(bc2.reshape(1, -1)), full(Wu1), full(bu1.reshape(1, -1)),
            full(Wu2), full(bu2.reshape(1, -1)), full(Wp1),
            full(bp1.reshape(1, -1)), full(Wp2.reshape(1, EMB_DIM)),
            full(bp2.reshape(1, 1)),
        ],
        out_specs=pl.BlockSpec((BLK, 1), lambda i: (i, 0)),
        out_shape=jax.ShapeDtypeStruct((B, 1), jnp.float32),
        compiler_params=pltpu.CompilerParams(
            dimension_semantics=("parallel",)),
    )(post, sd, mask, species, emb, Wc1, bc1.reshape(-1, 1), Wc2,
      bc2.reshape(1, -1), Wu1, bu1.reshape(1, -1), Wu2, bu2.reshape(1, -1),
      Wp1, bp1.reshape(1, -1), Wp2.reshape(1, EMB_DIM), bp2.reshape(1, 1))
    return out.reshape(B)
